# Initial kernel scaffold; baseline (speedup 1.0000x reference)
import functools

import jax
import jax.numpy as jnp
from jax import lax
from jax.experimental import pallas as pl
from jax.experimental.pallas import tpu as pltpu
from jax.experimental.pallas import tpu_sc as plsc

N = 10000
E = 320000
D = 128
DR = 16

NC = 2
NS = 16
NW = NC * NS

K = 80
EPW = E // NW
CHUNKS = EPW // K
RPT = N // NS
ZR = 125


def _filt_body(eb_ref, w_ref, b_ref, out_ref):
    out_ref[...] = lax.dot_general(
        eb_ref[...], w_ref[...], (((1,), (1,)), ((), ())),
        preferred_element_type=jnp.float32) + b_ref[...]


def _compute_filt(edge_basis, W, b):
    BE = 4000
    return pl.pallas_call(
        _filt_body,
        grid=(E // BE,),
        in_specs=[
            pl.BlockSpec((BE, DR), lambda i: (i, 0)),
            pl.BlockSpec((D, DR), lambda i: (0, 0)),
            pl.BlockSpec((1, D), lambda i: (0, 0)),
        ],
        out_specs=pl.BlockSpec((BE, D), lambda i: (i, 0)),
        out_shape=jax.ShapeDtypeStruct((E, D), jnp.float32),
    )(edge_basis, W, b.reshape(1, D))


def _sc_body(x_hbm, src_hbm, dst_hbm, filt_hbm, out_hbm,
             src_v, dst_v, rows_v, filt_v, zero_v, acc, sem_g, sem_f):
    cid = lax.axis_index("c")
    sid = lax.axis_index("s")
    wid = sid * NC + cid

    def zrow(r, carry):
        for c in range(D // 16):
            zero_v[r, pl.ds(c * 16, 16)] = jnp.zeros((16,), jnp.float32)
        return carry
    lax.fori_loop(0, ZR, zrow, 0)
    for t in range(RPT // ZR):
        pltpu.sync_copy(zero_v, acc.at[pl.ds(sid * RPT + t * ZR, ZR)])
    plsc.subcore_barrier()

    pltpu.sync_copy(src_hbm.at[pl.ds(wid * CHUNKS, CHUNKS)], src_v)
    pltpu.sync_copy(dst_hbm.at[pl.ds(wid * CHUNKS, CHUNKS)], dst_v)

    def chunk(j, carry):
        cp_g = pltpu.async_copy(x_hbm.at[src_v.at[j]], rows_v, sem_g)
        cp_f = pltpu.async_copy(
            filt_hbm.at[pl.ds((wid * CHUNKS + j) * K, K)], filt_v, sem_f)
        cp_g.wait()
        cp_f.wait()

        def mrow(e, c2):
            for c in range(D // 16):
                s = pl.ds(c * 16, 16)
                filt_v[e, s] = filt_v[e, s] * rows_v[e, s]
            return c2
        lax.fori_loop(0, K, mrow, 0)

        pltpu.sync_copy(filt_v, acc.at[dst_v.at[j]], add=True)
        return carry
    lax.fori_loop(0, CHUNKS, chunk, 0)
    plsc.subcore_barrier()

    pltpu.sync_copy(acc.at[pl.ds(sid * RPT, RPT)],
                    out_hbm.at[cid, pl.ds(sid * RPT, RPT)])


_sc_scatter = functools.partial(
    pl.kernel,
    out_type=jax.ShapeDtypeStruct((NC, N, D), jnp.float32),
    mesh=plsc.VectorSubcoreMesh(core_axis_name="c", subcore_axis_name="s"),
    scratch_types=[
        pltpu.VMEM((NW * CHUNKS, K), jnp.int32),
        pltpu.VMEM((NW * CHUNKS, K), jnp.int32),
        pltpu.VMEM((K, D), jnp.float32),
        pltpu.VMEM((K, D), jnp.float32),
        pltpu.VMEM((ZR, D), jnp.float32),
        pltpu.VMEM_SHARED((N, D), jnp.float32),
        pltpu.SemaphoreType.DMA,
        pltpu.SemaphoreType.DMA,
    ],
)(_sc_body)


def _add_body(p_ref, o_ref):
    o_ref[...] = p_ref[0] + p_ref[1]


def _combine(partials):
    BN = 2000
    return pl.pallas_call(
        _add_body,
        grid=(N // BN,),
        in_specs=[pl.BlockSpec((NC, BN, D), lambda i: (0, i, 0))],
        out_specs=pl.BlockSpec((BN, D), lambda i: (i, 0)),
        out_shape=jax.ShapeDtypeStruct((N, D), jnp.float32),
    )(partials)


def kernel(x, edge_index, edge_basis, W, b):
    filt = _compute_filt(edge_basis, W, b)
    src = edge_index[0].reshape(NW * CHUNKS, K)
    dst = edge_index[1].reshape(NW * CHUNKS, K)
    partials = _sc_scatter(x, src, dst, filt)
    return _combine(partials)


# speedup vs baseline: 1.9055x; 1.9055x over previous
import functools

import jax
import jax.numpy as jnp
from jax import lax
from jax.experimental import pallas as pl
from jax.experimental.pallas import tpu as pltpu
from jax.experimental.pallas import tpu_sc as plsc

N = 10000
E = 320000
D = 128
DH = D // 2
DR = 16

NC = 2
NS = 16

K = 80
EPT = E // NS
TCH = EPT // K
WPT = 624
TAIL = N - NS * WPT
ZR = 208


def _filt_body(eb_ref, w_ref, b_ref, out_ref):
    out_ref[0] = lax.dot_general(
        eb_ref[...], w_ref[0], (((1,), (1,)), ((), ())),
        preferred_element_type=jnp.float32) + b_ref[0]


def _compute_filt(edge_basis, W, b):
    BE = 4000
    return pl.pallas_call(
        _filt_body,
        grid=(NC, E // BE),
        in_specs=[
            pl.BlockSpec((BE, DR), lambda c, i: (i, 0)),
            pl.BlockSpec((1, DH, DR), lambda c, i: (c, 0, 0)),
            pl.BlockSpec((1, 1, DH), lambda c, i: (c, 0, 0)),
        ],
        out_specs=pl.BlockSpec((1, BE, DH), lambda c, i: (c, i, 0)),
        out_shape=jax.ShapeDtypeStruct((NC, E, DH), jnp.float32),
    )(edge_basis, W.reshape(NC, DH, DR), b.reshape(NC, 1, DH))


def _sc_body(x_hbm, src_hbm, dst_hbm, filt_hbm, out_hbm,
             src_v, dst_v, rows_v, filt_v, zero_v, acc, sem_g, sem_f):
    cid = lax.axis_index("c")
    sid = lax.axis_index("s")

    def zrow(r, carry):
        for c in range(DH // 16):
            zero_v[r, pl.ds(c * 16, 16)] = jnp.zeros((16,), jnp.float32)
        return carry
    lax.fori_loop(0, ZR, zrow, 0)
    for t in range(WPT // ZR):
        pltpu.sync_copy(zero_v, acc.at[pl.ds(sid * WPT + t * ZR, ZR)])

    @pl.when(sid == 0)
    def _zero_tail():
        pltpu.sync_copy(zero_v.at[pl.ds(0, TAIL)],
                        acc.at[pl.ds(NS * WPT, TAIL)])
    plsc.subcore_barrier()

    pltpu.sync_copy(src_hbm.at[sid], src_v)
    pltpu.sync_copy(dst_hbm.at[sid], dst_v)

    def chunk(j, carry):
        cp_g = pltpu.async_copy(x_hbm.at[cid].at[src_v.at[j]], rows_v, sem_g)
        cp_f = pltpu.async_copy(
            filt_hbm.at[cid, pl.ds(sid * EPT + j * K, K)], filt_v, sem_f)
        cp_g.wait()
        cp_f.wait()

        def mrow(e, c2):
            for c in range(DH // 16):
                s = pl.ds(c * 16, 16)
                filt_v[e, s] = filt_v[e, s] * rows_v[e, s]
            return c2
        lax.fori_loop(0, K, mrow, 0)

        pltpu.sync_copy(filt_v, acc.at[dst_v.at[j]], add=True)
        return carry
    lax.fori_loop(0, TCH, chunk, 0)
    plsc.subcore_barrier()

    pltpu.sync_copy(acc.at[pl.ds(sid * WPT, WPT)],
                    out_hbm.at[cid, pl.ds(sid * WPT, WPT)])

    @pl.when(sid == 0)
    def _write_tail():
        pltpu.sync_copy(acc.at[pl.ds(NS * WPT, TAIL)],
                        out_hbm.at[cid, pl.ds(NS * WPT, TAIL)])


_sc_scatter = functools.partial(
    pl.kernel,
    out_type=jax.ShapeDtypeStruct((NC, N, DH), jnp.float32),
    mesh=plsc.VectorSubcoreMesh(core_axis_name="c", subcore_axis_name="s"),
    compiler_params=pltpu.CompilerParams(use_tc_tiling_on_sc=False),
    scratch_types=[
        pltpu.VMEM((TCH, K), jnp.int32),
        pltpu.VMEM((TCH, K), jnp.int32),
        pltpu.VMEM((K, DH), jnp.float32),
        pltpu.VMEM((K, DH), jnp.float32),
        pltpu.VMEM((ZR, DH), jnp.float32),
        pltpu.VMEM_SHARED((N, DH), jnp.float32),
        pltpu.SemaphoreType.DMA,
        pltpu.SemaphoreType.DMA,
    ],
)(_sc_body)


def _cat_body(p_ref, o_ref):
    o_ref[:, :DH] = p_ref[0]
    o_ref[:, DH:] = p_ref[1]


def _concat(partials):
    BN = 2000
    return pl.pallas_call(
        _cat_body,
        grid=(N // BN,),
        in_specs=[pl.BlockSpec((NC, BN, DH), lambda i: (0, i, 0))],
        out_specs=pl.BlockSpec((BN, D), lambda i: (i, 0)),
        out_shape=jax.ShapeDtypeStruct((N, D), jnp.float32),
    )(partials)


def kernel(x, edge_index, edge_basis, W, b):
    filt = _compute_filt(edge_basis, W, b)
    x2 = x.reshape(N, NC, DH).transpose(1, 0, 2)
    src = edge_index[0].reshape(NS, TCH, K)
    dst = edge_index[1].reshape(NS, TCH, K)
    halves = _sc_scatter(x2, src, dst, filt)
    return _concat(halves)


# speedup vs baseline: 1.9950x; 1.0470x over previous
import functools

import jax
import jax.numpy as jnp
from jax import lax
from jax.experimental import pallas as pl
from jax.experimental.pallas import tpu as pltpu
from jax.experimental.pallas import tpu_sc as plsc

N = 10000
E = 320000
E2 = E // 2
D = 128
DH = D // 2
DR = 16

NC = 2
NS = 16

KH = 40
K = 2 * KH
EPT2 = E2 // NS
TCH = EPT2 // KH
RPT = N // NS
ZR = 125


def _filt_body(eba_ref, ebb_ref, w_ref, b_ref, out_ref):
    dn = (((1,), (1,)), ((), ()))
    out_ref[0, :, :DH] = lax.dot_general(
        eba_ref[...], w_ref[0], dn,
        preferred_element_type=jnp.float32) + b_ref[0]
    out_ref[0, :, DH:] = lax.dot_general(
        ebb_ref[...], w_ref[0], dn,
        preferred_element_type=jnp.float32) + b_ref[0]


def _compute_filt(edge_basis, W, b):
    BE = 4000
    nb = E2 // BE
    return pl.pallas_call(
        _filt_body,
        grid=(NC, nb),
        in_specs=[
            pl.BlockSpec((BE, DR), lambda c, i: (i, 0)),
            pl.BlockSpec((BE, DR), lambda c, i: (i + nb, 0)),
            pl.BlockSpec((1, DH, DR), lambda c, i: (c, 0, 0)),
            pl.BlockSpec((1, 1, DH), lambda c, i: (c, 0, 0)),
        ],
        out_specs=pl.BlockSpec((1, BE, D), lambda c, i: (c, i, 0)),
        out_shape=jax.ShapeDtypeStruct((NC, E2, D), jnp.float32),
    )(edge_basis, edge_basis, W.reshape(NC, DH, DR), b.reshape(NC, 1, DH))


def _sc_body(x_hbm, src_hbm, dst_hbm, filt_hbm, out_hbm, x2_hbm,
             src_v, dst_v, rows_v, filt_v, msg_v, zero_v, xst_v, xh_v,
             acc, sem_g, sem_f):
    cid = lax.axis_index("c")
    sid = lax.axis_index("s")

    def zrow(r, carry):
        for c in range(DH // 16):
            zero_v[r, pl.ds(c * 16, 16)] = jnp.zeros((16,), jnp.float32)
        return carry
    lax.fori_loop(0, ZR, zrow, 0)

    for t in range(RPT // ZR):
        r0 = sid * RPT + t * ZR
        pltpu.sync_copy(x_hbm.at[pl.ds(r0, ZR)], xst_v)

        def split(r, carry):
            for c in range(DH // 16):
                xh_v[r, pl.ds(c * 16, 16)] = (
                    xst_v[r, pl.ds(cid * DH + c * 16, 16)])
            return carry
        lax.fori_loop(0, ZR, split, 0)
        pltpu.sync_copy(xh_v, x2_hbm.at[cid, pl.ds(r0, ZR)])
        pltpu.sync_copy(zero_v, acc.at[pl.ds(r0, ZR)])
    plsc.subcore_barrier()

    pltpu.sync_copy(src_hbm.at[sid], src_v)
    pltpu.sync_copy(dst_hbm.at[sid], dst_v)

    def chunk(j, carry):
        cp_g = pltpu.async_copy(x2_hbm.at[cid].at[src_v.at[j]], rows_v, sem_g)
        cp_f = pltpu.async_copy(
            filt_hbm.at[cid, pl.ds(sid * EPT2 + j * KH, KH)], filt_v, sem_f)
        cp_g.wait()
        cp_f.wait()

        def mrow(r, c2):
            for c in range(DH // 16):
                s = pl.ds(c * 16, 16)
                msg_v[2 * r, s] = filt_v[r, s] * rows_v[r, s]
                msg_v[2 * r + 1, s] = (
                    filt_v[r, pl.ds(DH + c * 16, 16)] * rows_v[KH + r, s])
            return c2
        lax.fori_loop(0, KH, mrow, 0)

        pltpu.sync_copy(msg_v, acc.at[dst_v.at[j]], add=True)
        return carry
    lax.fori_loop(0, TCH, chunk, 0)
    plsc.subcore_barrier()

    pltpu.sync_copy(acc.at[pl.ds(sid * RPT, RPT)],
                    out_hbm.at[pl.ds(sid * RPT, RPT), pl.ds(cid * DH, DH)])


_sc_scatter = functools.partial(
    pl.kernel,
    out_type=(
        jax.ShapeDtypeStruct((N, D), jnp.float32),
        jax.ShapeDtypeStruct((NC, N, DH), jnp.float32),
    ),
    mesh=plsc.VectorSubcoreMesh(core_axis_name="c", subcore_axis_name="s"),
    compiler_params=pltpu.CompilerParams(use_tc_tiling_on_sc=False),
    scratch_types=[
        pltpu.VMEM((TCH, K), jnp.int32),
        pltpu.VMEM((TCH, K), jnp.int32),
        pltpu.VMEM((K, DH), jnp.float32),
        pltpu.VMEM((KH, D), jnp.float32),
        pltpu.VMEM((K, DH), jnp.float32),
        pltpu.VMEM((ZR, DH), jnp.float32),
        pltpu.VMEM((ZR, D), jnp.float32),
        pltpu.VMEM((ZR, DH), jnp.float32),
        pltpu.VMEM_SHARED((N, DH), jnp.float32),
        pltpu.SemaphoreType.DMA,
        pltpu.SemaphoreType.DMA,
    ],
)(_sc_body)


def kernel(x, edge_index, edge_basis, W, b):
    filt2 = _compute_filt(edge_basis, W, b)
    src = edge_index[0]
    dst = edge_index[1]
    srcA = src[:E2].reshape(NS, TCH, KH)
    srcB = src[E2:].reshape(NS, TCH, KH)
    src_arr = jnp.concatenate([srcA, srcB], axis=2)
    dstA = dst[:E2].reshape(NS, TCH, KH)
    dstB = dst[E2:].reshape(NS, TCH, KH)
    dst_arr = jnp.stack([dstA, dstB], axis=3).reshape(NS, TCH, K)
    out, _ = _sc_scatter(x, src_arr, dst_arr, filt2)
    return out


# speedup vs baseline: 2.5786x; 1.2925x over previous
import functools

import jax
import jax.numpy as jnp
from jax import lax
from jax.experimental import pallas as pl
from jax.experimental.pallas import tpu as pltpu
from jax.experimental.pallas import tpu_sc as plsc

N = 10000
E = 320000
E2 = E // 2
D = 128
DH = D // 2
DR = 16

NC = 2
NS = 16

KH = 40
K = 2 * KH
EPT2 = E2 // NS
TCH = EPT2 // KH
RPT = N // NS
XR = 100


def _filt_body(eba_ref, ebb_ref, w_ref, b_ref, out_ref):
    dn = (((1,), (1,)), ((), ()))
    out_ref[0, :, :DH] = lax.dot_general(
        eba_ref[...], w_ref[0], dn,
        preferred_element_type=jnp.float32) + b_ref[0]
    out_ref[0, :, DH:] = lax.dot_general(
        ebb_ref[...], w_ref[0], dn,
        preferred_element_type=jnp.float32) + b_ref[0]


def _compute_filt(edge_basis, W, b):
    BE = 4000
    nb = E2 // BE
    return pl.pallas_call(
        _filt_body,
        grid=(NC, nb),
        in_specs=[
            pl.BlockSpec((BE, DR), lambda c, i: (i, 0)),
            pl.BlockSpec((BE, DR), lambda c, i: (i + nb, 0)),
            pl.BlockSpec((1, DH, DR), lambda c, i: (c, 0, 0)),
            pl.BlockSpec((1, 1, DH), lambda c, i: (c, 0, 0)),
        ],
        out_specs=pl.BlockSpec((1, BE, D), lambda c, i: (c, i, 0)),
        out_shape=jax.ShapeDtypeStruct((NC, E2, D), jnp.float32),
    )(edge_basis, edge_basis, W.reshape(NC, DH, DR), b.reshape(NC, 1, DH))


def _sc_body(x_hbm, src_hbm, dst_hbm, filt_hbm, out_hbm, x2_hbm,
             src_v, dst_v, rows0_v, rows1_v, filt0_v, filt1_v, msg_v,
             xst_v, xh_v, acc,
             sem_g0, sem_f0, sem_g1, sem_f1):
    cid = lax.axis_index("c")
    sid = lax.axis_index("s")

    def zrow(r, carry):
        for c in range(DH // 16):
            msg_v[r, pl.ds(c * 16, 16)] = jnp.zeros((16,), jnp.float32)
        return carry
    lax.fori_loop(0, K, zrow, 0)
    for t in range(RPT // K):
        pltpu.sync_copy(msg_v, acc.at[pl.ds(sid * RPT + t * K, K)])
    ztail = RPT - (RPT // K) * K
    pltpu.sync_copy(msg_v.at[pl.ds(0, ztail)],
                    acc.at[pl.ds(sid * RPT + (RPT // K) * K, ztail)])

    def make_split(col0, nr):
        def split(r, carry):
            for c in range(DH // 16):
                xh_v[r, pl.ds(c * 16, 16)] = xst_v[r, pl.ds(col0 + c * 16, 16)]
            return carry
        return split

    for t in range(RPT // XR + 1):
        nr = XR if t < RPT // XR else RPT - (RPT // XR) * XR
        r0 = sid * RPT + t * XR
        pltpu.sync_copy(x_hbm.at[pl.ds(r0, nr)], xst_v.at[pl.ds(0, nr)])

        @pl.when(cid == 0)
        def _lo():
            lax.fori_loop(0, nr, make_split(0, nr), 0)

        @pl.when(cid == 1)
        def _hi():
            lax.fori_loop(0, nr, make_split(DH, nr), 0)
        pltpu.sync_copy(xh_v.at[pl.ds(0, nr)], x2_hbm.at[cid, pl.ds(r0, nr)])
    plsc.subcore_barrier()

    pltpu.sync_copy(src_hbm.at[sid], src_v)
    pltpu.sync_copy(dst_hbm.at[sid], dst_v)

    fbase = sid * EPT2

    def start(j, rows_v, filt_v, sem_g, sem_f):
        cp_g = pltpu.async_copy(x2_hbm.at[cid].at[src_v.at[j]], rows_v, sem_g)
        cp_f = pltpu.async_copy(
            filt_hbm.at[cid, pl.ds(fbase + j * KH, KH)], filt_v, sem_f)
        return cp_g, cp_f

    def finish(j, cps, rows_v, filt_v):
        cps[0].wait()
        cps[1].wait()

        def mrow(r, c2):
            for c in range(DH // 16):
                s = pl.ds(c * 16, 16)
                msg_v[r, s] = filt_v[r, s] * rows_v[r, s]
            for c in range(DH // 16):
                s = pl.ds(c * 16, 16)
                msg_v[KH + r, s] = (
                    filt_v[r, pl.ds(DH + c * 16, 16)] * rows_v[KH + r, s])
            return c2
        lax.fori_loop(0, KH, mrow, 0)

        pltpu.sync_copy(msg_v, acc.at[dst_v.at[j]], add=True)

    start(0, rows0_v, filt0_v, sem_g0, sem_f0)

    def pipe(j2, carry):
        a = 2 * j2
        start(a + 1, rows1_v, filt1_v, sem_g1, sem_f1)
        cps0 = (pltpu.make_async_copy(x2_hbm.at[cid].at[src_v.at[a]],
                                      rows0_v, sem_g0),
                pltpu.make_async_copy(filt_hbm.at[cid, pl.ds(fbase, KH)],
                                      filt0_v, sem_f0))
        finish(a, cps0, rows0_v, filt0_v)
        nxt = jnp.minimum(a + 2, TCH - 1)
        start(nxt, rows0_v, filt0_v, sem_g0, sem_f0)
        cps1 = (pltpu.make_async_copy(x2_hbm.at[cid].at[src_v.at[a + 1]],
                                      rows1_v, sem_g1),
                pltpu.make_async_copy(filt_hbm.at[cid, pl.ds(fbase, KH)],
                                      filt1_v, sem_f1))
        finish(a + 1, cps1, rows1_v, filt1_v)
        return carry
    lax.fori_loop(0, TCH // 2, pipe, 0)
    pltpu.make_async_copy(x2_hbm.at[cid].at[src_v.at[0]],
                          rows0_v, sem_g0).wait()
    pltpu.make_async_copy(filt_hbm.at[cid, pl.ds(fbase, KH)],
                          filt0_v, sem_f0).wait()
    plsc.subcore_barrier()

    pltpu.sync_copy(acc.at[pl.ds(sid * RPT, RPT)],
                    out_hbm.at[pl.ds(sid * RPT, RPT), pl.ds(cid * DH, DH)])


_sc_scatter = functools.partial(
    pl.kernel,
    out_type=(
        jax.ShapeDtypeStruct((N, D), jnp.float32),
        jax.ShapeDtypeStruct((NC, N, DH), jnp.float32),
    ),
    mesh=plsc.VectorSubcoreMesh(core_axis_name="c", subcore_axis_name="s"),
    compiler_params=pltpu.CompilerParams(use_tc_tiling_on_sc=False),
    scratch_types=[
        pltpu.VMEM((TCH, K), jnp.int32),
        pltpu.VMEM((TCH, K), jnp.int32),
        pltpu.VMEM((K, DH), jnp.float32),
        pltpu.VMEM((K, DH), jnp.float32),
        pltpu.VMEM((KH, D), jnp.float32),
        pltpu.VMEM((KH, D), jnp.float32),
        pltpu.VMEM((K, DH), jnp.float32),
        pltpu.VMEM((XR, D), jnp.float32),
        pltpu.VMEM((XR, DH), jnp.float32),
        pltpu.VMEM_SHARED((N, DH), jnp.float32),
        pltpu.SemaphoreType.DMA,
        pltpu.SemaphoreType.DMA,
        pltpu.SemaphoreType.DMA,
        pltpu.SemaphoreType.DMA,
    ],
)(_sc_body)


def kernel(x, edge_index, edge_basis, W, b):
    filt2 = _compute_filt(edge_basis, W, b)
    src = edge_index[0]
    dst = edge_index[1]
    srcA = src[:E2].reshape(NS, TCH, KH)
    srcB = src[E2:].reshape(NS, TCH, KH)
    src_arr = jnp.concatenate([srcA, srcB], axis=2)
    dstA = dst[:E2].reshape(NS, TCH, KH)
    dstB = dst[E2:].reshape(NS, TCH, KH)
    dst_arr = jnp.concatenate([dstA, dstB], axis=2)
    out, _ = _sc_scatter(x, src_arr, dst_arr, filt2)
    return out
